# speedup vs baseline: 1.5938x; 1.5938x over previous
import functools

import jax
import jax.numpy as jnp
from jax import lax
from jax.experimental import pallas as pl
from jax.experimental.pallas import tpu as pltpu
from jax.experimental.pallas import tpu_sc as plsc

_D = 4 * 64 * 64
_B = 1024
_NC = 2
_NS = 16
_NW = _NC * _NS
_BPW = _B // _NW
_CH = 4
_NCH = _BPW // _CH

_mesh = plsc.VectorSubcoreMesh(core_axis_name="c", subcore_axis_name="s")


@functools.partial(
    pl.kernel,
    mesh=_mesh,
    out_type=jax.ShapeDtypeStruct((_B, _D), jnp.float32),
    scratch_types=[
        pltpu.VMEM((_NCH, _CH), jnp.int32),
        pltpu.VMEM((_CH, _D), jnp.float32),
        pltpu.SemaphoreType.DMA,
    ],
)
def _emb_gather(idx_hbm, table_hbm, out_hbm, idx_v, rows_v, sem):
    wid = lax.axis_index("s") * _NC + lax.axis_index("c")
    pltpu.sync_copy(idx_hbm.at[wid], idx_v)
    for c in range(_NCH):
        pltpu.async_copy(table_hbm.at[idx_v.at[c]], rows_v, sem).wait()
        pltpu.sync_copy(rows_v, out_hbm.at[pl.ds(wid * _BPW + c * _CH, _CH)])


def kernel(x, table):
    idx = x.astype(jnp.int32).reshape(_NW, _NCH, _CH)
    out = _emb_gather(idx, table)
    return out.reshape(_B, 4, 64, 64)


# speedup vs baseline: 1.6435x; 1.0312x over previous
import functools

import jax
import jax.numpy as jnp
from jax import lax
from jax.experimental import pallas as pl
from jax.experimental.pallas import tpu as pltpu
from jax.experimental.pallas import tpu_sc as plsc

_D = 4 * 64 * 64
_B = 1024
_NC = 2
_NS = 16
_NW = _NC * _NS
_BPW = _B // _NW
_CH = 2
_NCH = _BPW // _CH

_mesh = plsc.VectorSubcoreMesh(core_axis_name="c", subcore_axis_name="s")


@functools.partial(
    pl.kernel,
    mesh=_mesh,
    out_type=jax.ShapeDtypeStruct((_B, _D), jnp.float32),
    scratch_types=[
        pltpu.VMEM((_NCH, _CH), jnp.int32),
        pltpu.VMEM((2, _CH, _D), jnp.float32),
        pltpu.SemaphoreType.DMA,
        pltpu.SemaphoreType.DMA,
        pltpu.SemaphoreType.DMA,
        pltpu.SemaphoreType.DMA,
    ],
)
def _emb_gather(idx_hbm, table_hbm, out_hbm, idx_v, rows_v,
                s_in0, s_in1, s_out0, s_out1):
    wid = lax.axis_index("s") * _NC + lax.axis_index("c")
    base = wid * _BPW
    pltpu.sync_copy(idx_hbm.at[wid], idx_v)
    s_in = (s_in0, s_in1)
    s_out = (s_out0, s_out1)

    def gather(c):
        b = c % 2
        return pltpu.make_async_copy(
            table_hbm.at[idx_v.at[c]], rows_v.at[b], s_in[b])

    def put(c):
        b = c % 2
        return pltpu.make_async_copy(
            rows_v.at[b], out_hbm.at[pl.ds(base + c * _CH, _CH)], s_out[b])

    gather(0).start()
    gather(1).start()
    for c in range(_NCH):
        gather(c).wait()
        put(c).start()
        if c + 2 < _NCH:
            put(c).wait()
            gather(c + 2).start()
    put(_NCH - 2).wait()
    put(_NCH - 1).wait()


def kernel(x, table):
    idx = x.astype(jnp.int32).reshape(_NW, _NCH, _CH)
    out = _emb_gather(idx, table)
    return out.reshape(_B, 4, 64, 64)
